# baseline (speedup 1.0000x reference)
import functools
import math

import jax
import jax.numpy as jnp
from jax import lax
from jax.experimental import pallas as pl
from jax.experimental.pallas import tpu as pltpu
from jax.experimental.pallas import tpu_sc as plsc

_NC = 2
_NS = 16
_LANES = 16
_CW = 128
_CHUNK = 64


def _zero_vmem(ref, rows, cols):
    zero = jnp.zeros((_LANES,), jnp.float32)

    def body(r, _):
        for cb in range(cols // _LANES):
            ref[r, pl.ds(cb * _LANES, _LANES)] = zero
        return 0

    lax.fori_loop(0, rows, body, 0)


def _build_scatter_idx(src_ref, dst_ref, off, base, half):
    dump = jnp.full((_LANES,), half, jnp.int32)
    for k in range(_CHUNK // _LANES):
        v = src_ref[pl.ds(off + k * _LANES, _LANES)] - base
        oob = (v < 0) | (v >= half)
        dst_ref[pl.ds(k * _LANES, _LANES)] = jnp.where(oob, dump, v)


def _sc_token2map(x_pad, idx_gather, idx_scatter, B, N, P):
    CH = 128
    toks_per_tile = N // _NS
    nchunks = toks_per_tile // CH
    half = P // _NC
    rows_tile = half // _NS
    trows = half + _LANES

    mesh = plsc.VectorSubcoreMesh(core_axis_name="c", subcore_axis_name="s")

    @functools.partial(
        pl.kernel,
        out_type=jax.ShapeDtypeStruct((B, P, _CW), jnp.float32),
        mesh=mesh,
        scratch_types=[
            pltpu.VMEM_SHARED((trows, _CW), jnp.float32),
            pltpu.VMEM((toks_per_tile,), jnp.int32),
            pltpu.VMEM((toks_per_tile,), jnp.int32),
            pltpu.VMEM((2, CH), jnp.int32),
            pltpu.VMEM((2, CH, _CW), jnp.float32),
            pltpu.VMEM((32, _CW), jnp.float32),
            pltpu.SemaphoreType.DMA,
            pltpu.SemaphoreType.DMA,
            pltpu.SemaphoreType.DMA,
            pltpu.SemaphoreType.DMA,
        ],
    )
    def body(x_hbm, ig_hbm, is_hbm, out_hbm, tab_sh, ig_all, is_all, is_c,
             gbuf, zbuf, sem0, sem1, sem2, sem3):
        cid = lax.axis_index("c")
        sid = lax.axis_index("s")
        base = cid * half
        row0 = sid * rows_tile
        tok0 = sid * toks_per_tile
        sems = (sem0, sem1)
        ssems = (sem2, sem3)

        _zero_vmem(zbuf, 32, _CW)

        def fetch(ch, slot):
            idx = ig_all.at[pl.ds(ch * CH, CH)]
            return pltpu.async_copy(x_hbm.at[idx], gbuf.at[slot],
                                    sems[slot])

        for b in range(B):
            for z in range(rows_tile // 32):
                pltpu.sync_copy(zbuf, tab_sh.at[pl.ds(row0 + z * 32, 32)])
            pltpu.sync_copy(ig_hbm.at[b, pl.ds(tok0, toks_per_tile)], ig_all)
            pltpu.sync_copy(is_hbm.at[b, pl.ds(tok0, toks_per_tile)], is_all)
            plsc.subcore_barrier()

            dma = fetch(0, 0)
            sdma = [None, None]
            for ch in range(nchunks):
                slot = ch % 2
                dma.wait()
                if ch + 1 < nchunks:
                    if sdma[1 - slot] is not None:
                        sdma[1 - slot].wait()
                    dma = fetch(ch + 1, 1 - slot)
                for k2 in range(CH // _CHUNK):
                    _build_scatter_idx(
                        is_all, is_c.at[slot].at[pl.ds(k2 * _CHUNK, _CHUNK)],
                        ch * CH + k2 * _CHUNK, base, half)
                sdma[slot] = pltpu.async_copy(
                    gbuf.at[slot], tab_sh.at[is_c.at[slot]], ssems[slot],
                    add=True)
            for s in sdma:
                if s is not None:
                    s.wait()
            plsc.subcore_barrier()

            pltpu.sync_copy(tab_sh.at[pl.ds(row0, rows_tile)],
                            out_hbm.at[b, pl.ds(base + row0, rows_tile)])

    return body(x_pad, idx_gather, idx_scatter)


def _sc_map2token(map_pad, idx_gather, idx_scatter, w_bc, B, N):
    CH = 128
    toks_per_tile = N // _NS
    nchunks = toks_per_tile // CH
    half = N // _NC
    rows_tile = half // _NS
    trows = half + 8

    mesh = plsc.VectorSubcoreMesh(core_axis_name="c", subcore_axis_name="s")

    @functools.partial(
        pl.kernel,
        out_type=jax.ShapeDtypeStruct((B, N, _CW), jnp.float32),
        mesh=mesh,
        scratch_types=[
            pltpu.VMEM_SHARED((trows, _CW), jnp.float32),
            pltpu.VMEM((toks_per_tile,), jnp.int32),
            pltpu.VMEM((toks_per_tile,), jnp.int32),
            pltpu.VMEM((2, CH), jnp.int32),
            pltpu.VMEM((2, CH, _CW), jnp.float32),
            pltpu.VMEM((CH, _LANES), jnp.float32),
            pltpu.SemaphoreType.DMA,
            pltpu.SemaphoreType.DMA,
            pltpu.SemaphoreType.DMA,
            pltpu.SemaphoreType.DMA,
        ],
    )
    def body(m_hbm, ig_hbm, is_hbm, w_hbm, out_hbm, tab_sh, ig_all, is_all,
             is_c, gbuf, w_all, sem0, sem1, sem2, sem3):
        cid = lax.axis_index("c")
        sid = lax.axis_index("s")
        base = cid * half
        row0 = sid * rows_tile
        tok0 = sid * toks_per_tile
        sems = (sem0, sem1)
        ssems = (sem2, sem3)
        zrows = 16

        def fetch(ch, slot):
            idx = ig_all.at[pl.ds(ch * CH, CH)]
            return pltpu.async_copy(m_hbm.at[idx], gbuf.at[slot],
                                    sems[slot])

        for b in range(B):
            _zero_vmem(gbuf.at[1], zrows, _CW)
            for z in range(rows_tile // zrows):
                pltpu.sync_copy(gbuf.at[1, pl.ds(0, zrows)],
                                tab_sh.at[pl.ds(row0 + z * zrows, zrows)])
            pltpu.sync_copy(ig_hbm.at[b, pl.ds(tok0, toks_per_tile)], ig_all)
            pltpu.sync_copy(is_hbm.at[b, pl.ds(tok0, toks_per_tile)], is_all)
            plsc.subcore_barrier()

            dma = fetch(0, 0)
            sdma = [None, None]
            for ch in range(nchunks):
                slot = ch % 2
                pltpu.sync_copy(w_hbm.at[b, pl.ds(tok0 + ch * CH, CH)],
                                w_all)
                dma.wait()
                if ch + 1 < nchunks:
                    if sdma[1 - slot] is not None:
                        sdma[1 - slot].wait()
                    dma = fetch(ch + 1, 1 - slot)
                for k2 in range(CH // _CHUNK):
                    _build_scatter_idx(
                        is_all, is_c.at[slot].at[pl.ds(k2 * _CHUNK, _CHUNK)],
                        ch * CH + k2 * _CHUNK, base, half)
                @plsc.parallel_loop(0, CH, unroll=4)
                def wmul(t):
                    wv = w_all[t, :]
                    for cb in range(_CW // _LANES):
                        sl = pl.ds(cb * _LANES, _LANES)
                        gbuf[slot, t, sl] = gbuf[slot, t, sl] * wv

                sdma[slot] = pltpu.async_copy(
                    gbuf.at[slot], tab_sh.at[is_c.at[slot]], ssems[slot],
                    add=True)
            for s in sdma:
                if s is not None:
                    s.wait()
            plsc.subcore_barrier()

            pltpu.sync_copy(tab_sh.at[pl.ds(row0, rows_tile)],
                            out_hbm.at[b, pl.ds(base + row0, rows_tile)])

    return body(map_pad, idx_gather, idx_scatter, w_bc)


_PAD = 72

_TAPS = (
    (3, 65, True, 0),
    (2, 64, False, 1),
    (3, 64, False, 2),
    (1, 1, True, 3),
    (0, 0, False, 4),
    (1, 0, False, 5),
    (3, 1, True, 6),
    (2, 0, False, 7),
    (3, 0, False, 8),
)


def _tc_conv(table, w9, bias, B, C, Cout, Po):
    def body(tab_ref, w_ref, b_ref, out_ref, planes_ref):
        for q in range(4):
            sl = pl.ds(q * Po, Po)
            acc = tab_ref[0, sl, 0:C]
            c = tab_ref[0, sl, C:C + 1]
            planes_ref[q, pl.ds(_PAD, Po), :] = acc / (c + 1e-6)
            planes_ref[q, pl.ds(0, _PAD), :] = jnp.zeros((_PAD, C),
                                                         jnp.float32)

        colmask = (lax.broadcasted_iota(jnp.int32, (Po, 1), 0) % 64) != 0
        out = jnp.broadcast_to(b_ref[0][None, :], (Po, Cout))
        for (q, sh, msk, k) in _TAPS:
            x = planes_ref[q, pl.ds(_PAD - sh, Po), :]
            if msk:
                x = jnp.where(colmask, x, 0.0)
            out = out + jnp.dot(x, w_ref[k],
                                preferred_element_type=jnp.float32)
        lane = lax.broadcasted_iota(jnp.int32, (Po, _CW - Cout), 1)
        marker = jnp.where(lane == 0, 1.0, 0.0)
        out_ref[0, :, 0:Cout] = out
        out_ref[0, :, Cout:_CW] = marker

    return pl.pallas_call(
        body,
        grid=(B,),
        in_specs=[
            pl.BlockSpec((1, 4 * Po, _CW), lambda b: (b, 0, 0)),
            pl.BlockSpec((9, C, Cout), lambda b: (0, 0, 0)),
            pl.BlockSpec((1, Cout), lambda b: (0, 0)),
        ],
        out_specs=pl.BlockSpec((1, Po, _CW), lambda b: (b, 0, 0)),
        out_shape=jax.ShapeDtypeStruct((B, Po, _CW), jnp.float32),
        scratch_shapes=[pltpu.VMEM((4, _PAD + Po, C), jnp.float32)],
    )(table, w9, bias)


def _tc_final(xt3, table2, skip_wT, params, prev, g, B, N, C, Cout):
    def body(x_ref, t_ref, sw_ref, p_ref, *rest):
        out_ref, conf_ref = rest[-2], rest[-1]
        acc_t = jnp.transpose(t_ref[0, :, 0:Cout])
        nrm_t = jnp.transpose(t_ref[0, :, Cout:Cout + 1])
        xt = acc_t / (nrm_t + 1e-6)
        xt = xt + jnp.dot(sw_ref[...], x_ref[0],
                          preferred_element_type=jnp.float32)
        xt = xt * p_ref[0][:, None] + p_ref[1][:, None]
        conf = jnp.sum(xt * p_ref[2][:, None], axis=0, keepdims=True) \
            + p_ref[3, 0]
        out_ref[0] = jnp.maximum(xt, 0.0)
        conf_ref[0] = conf

    RD = 2048
    in_specs = [
        pl.BlockSpec((1, C, RD), lambda b, r: (2 * b + g, 0, r)),
        pl.BlockSpec((1, RD, _CW), lambda b, r: (b, r, 0)),
        pl.BlockSpec((Cout, C), lambda b, r: (0, 0)),
        pl.BlockSpec((4, Cout), lambda b, r: (0, 0)),
    ]
    inputs = [xt3, table2, skip_wT, params]
    aliases = {}
    if prev is not None:
        in_specs += [pl.BlockSpec(memory_space=pl.ANY),
                     pl.BlockSpec(memory_space=pl.ANY)]
        inputs += [prev[0], prev[1]]
        aliases = {4: 0, 5: 1}
    return pl.pallas_call(
        body,
        grid=(B // 2, N // RD),
        in_specs=in_specs,
        out_specs=[
            pl.BlockSpec((1, Cout, RD), lambda b, r: (2 * b + g, 0, r)),
            pl.BlockSpec((1, 1, RD), lambda b, r: (2 * b + g, 0, r)),
        ],
        out_shape=[
            jax.ShapeDtypeStruct((B, Cout, N), jnp.float32),
            jax.ShapeDtypeStruct((B, 1, N), jnp.float32),
        ],
        input_output_aliases=aliases,
    )(*inputs)


def kernel(x, loc_orig, idx_agg, agg_weight, H, W, conv_w, conv_b, skip_w,
           bn_gamma, bn_beta, conf_w, conf_b):
    B, N, C = x.shape
    Cout = conv_w.shape[0]
    Hs = Ws = int(math.isqrt(N))
    Ho, Wo = Hs // 2, Ws // 2

    scale = jnp.array([W, H], dtype=jnp.float32)
    loc = jnp.rint(0.5 * (loc_orig + 1.0) * scale - 0.5).astype(jnp.int32)
    ix = jnp.clip(loc[..., 0], 0, W - 1)
    iy = jnp.clip(loc[..., 1], 0, H - 1)
    idxp = ((iy & 1) * (2 * Ho * Wo) + (ix & 1) * (Ho * Wo)
            + (iy >> 1) * Wo + (ix >> 1)).astype(jnp.int32)

    scale2 = jnp.array([Wo, Ho], dtype=jnp.float32)
    loc2 = jnp.rint(0.5 * (loc_orig + 1.0) * scale2 - 0.5).astype(jnp.int32)
    ix2 = jnp.clip(loc2[..., 0], 0, Wo - 1)
    iy2 = jnp.clip(loc2[..., 1], 0, Ho - 1)
    idx2 = (iy2 * Wo + ix2).astype(jnp.int32)

    ia = idx_agg.astype(jnp.int32)
    ia_off = ia + (jnp.arange(B, dtype=jnp.int32) * N)[:, None]
    idx2_off = idx2 + (jnp.arange(B, dtype=jnp.int32) * (Ho * Wo))[:, None]

    x_pad = jnp.concatenate([
        x,
        jnp.ones((B, N, 1), jnp.float32),
        jnp.zeros((B, N, _CW - C - 1), jnp.float32),
    ], axis=2).reshape(B * N, _CW)
    xt3 = x.transpose(0, 2, 1)
    skip_wT = skip_w.T
    w9 = conv_w.transpose(2, 3, 1, 0).reshape(9, C, Cout)
    bias = conv_b.reshape(1, Cout)
    w_bc = jnp.broadcast_to(agg_weight, (B, N, _LANES))
    inv = 1.0 / jnp.sqrt(jnp.float32(1.0 + 1e-5))
    params = jnp.stack([
        bn_gamma * inv,
        bn_beta,
        conf_w[:, 0],
        jnp.broadcast_to(conf_b, (Cout,)),
    ])

    PG = 2
    prev = None
    for g in range(B // PG):
        sl = slice(g, None, 2)
        table1 = _sc_token2map(x_pad, ia_off[sl], idxp[sl], PG, N, Hs * Ws)
        map2 = _tc_conv(table1, w9, bias, PG, C, Cout, Ho * Wo)
        m_pad = map2.reshape(PG * Ho * Wo, _CW)
        idx2_g = idx2[sl] + (jnp.arange(PG, dtype=jnp.int32)
                             * (Ho * Wo))[:, None]
        table2 = _sc_map2token(m_pad, idx2_g, ia[sl], w_bc[sl], PG, N)
        prev = _tc_final(xt3, table2, skip_wT, params, prev, g,
                         B, N, C, Cout)

    return prev[0].transpose(0, 2, 1), prev[1].transpose(0, 2, 1)


# speedup vs baseline: 1.0127x; 1.0127x over previous
import functools
import math

import jax
import jax.numpy as jnp
from jax import lax
from jax.experimental import pallas as pl
from jax.experimental.pallas import tpu as pltpu
from jax.experimental.pallas import tpu_sc as plsc

_NC = 2
_NS = 16
_LANES = 16
_CW = 128
_CHUNK = 64


def _zero_vmem(ref, rows, cols):
    zero = jnp.zeros((_LANES,), jnp.float32)

    def body(r, _):
        for cb in range(cols // _LANES):
            ref[r, pl.ds(cb * _LANES, _LANES)] = zero
        return 0

    lax.fori_loop(0, rows, body, 0)


def _build_scatter_idx(src_ref, dst_ref, off, base, half):
    dump = jnp.full((_LANES,), half, jnp.int32)
    for k in range(_CHUNK // _LANES):
        v = src_ref[pl.ds(off + k * _LANES, _LANES)] - base
        oob = (v < 0) | (v >= half)
        dst_ref[pl.ds(k * _LANES, _LANES)] = jnp.where(oob, dump, v)


def _sc_token2map(x_pad, idx_gather, idx_scatter, B, N, P):
    CH = 128
    toks_per_tile = N // _NS
    nchunks = toks_per_tile // CH
    half = P // _NC
    rows_tile = half // _NS
    trows = half + _LANES

    mesh = plsc.VectorSubcoreMesh(core_axis_name="c", subcore_axis_name="s")

    @functools.partial(
        pl.kernel,
        out_type=jax.ShapeDtypeStruct((B, P, _CW), jnp.float32),
        mesh=mesh,
        scratch_types=[
            pltpu.VMEM_SHARED((trows, _CW), jnp.float32),
            pltpu.VMEM((toks_per_tile,), jnp.int32),
            pltpu.VMEM((toks_per_tile,), jnp.int32),
            pltpu.VMEM((2, CH), jnp.int32),
            pltpu.VMEM((2, CH, _CW), jnp.float32),
            pltpu.VMEM((32, _CW), jnp.float32),
            pltpu.SemaphoreType.DMA,
            pltpu.SemaphoreType.DMA,
            pltpu.SemaphoreType.DMA,
            pltpu.SemaphoreType.DMA,
        ],
    )
    def body(x_hbm, ig_hbm, is_hbm, out_hbm, tab_sh, ig_all, is_all, is_c,
             gbuf, zbuf, sem0, sem1, sem2, sem3):
        cid = lax.axis_index("c")
        sid = lax.axis_index("s")
        base = cid * half
        row0 = sid * rows_tile
        tok0 = sid * toks_per_tile
        sems = (sem0, sem1)
        ssems = (sem2, sem3)

        _zero_vmem(zbuf, 32, _CW)

        def fetch(ch, slot):
            idx = ig_all.at[pl.ds(ch * CH, CH)]
            return pltpu.async_copy(x_hbm.at[idx], gbuf.at[slot],
                                    sems[slot])

        for b in range(B):
            for z in range(rows_tile // 32):
                pltpu.sync_copy(zbuf, tab_sh.at[pl.ds(row0 + z * 32, 32)])
            pltpu.sync_copy(ig_hbm.at[b, pl.ds(tok0, toks_per_tile)], ig_all)
            pltpu.sync_copy(is_hbm.at[b, pl.ds(tok0, toks_per_tile)], is_all)
            plsc.subcore_barrier()

            dma = fetch(0, 0)
            sdma = [None, None]
            for ch in range(nchunks):
                slot = ch % 2
                dma.wait()
                if ch + 1 < nchunks:
                    if sdma[1 - slot] is not None:
                        sdma[1 - slot].wait()
                    dma = fetch(ch + 1, 1 - slot)
                for k2 in range(CH // _CHUNK):
                    _build_scatter_idx(
                        is_all, is_c.at[slot].at[pl.ds(k2 * _CHUNK, _CHUNK)],
                        ch * CH + k2 * _CHUNK, base, half)
                sdma[slot] = pltpu.async_copy(
                    gbuf.at[slot], tab_sh.at[is_c.at[slot]], ssems[slot],
                    add=True)
            for s in sdma:
                if s is not None:
                    s.wait()
            plsc.subcore_barrier()

            pltpu.sync_copy(tab_sh.at[pl.ds(row0, rows_tile)],
                            out_hbm.at[b, pl.ds(base + row0, rows_tile)])

    return body(x_pad, idx_gather, idx_scatter)


def _sc_map2token(map_pad, idx_gather, idx_scatter, w_bc, B, N):
    CH = 128
    toks_per_tile = N // _NS
    nchunks = toks_per_tile // CH
    half = N // _NC
    rows_tile = half // _NS
    trows = half + 8

    mesh = plsc.VectorSubcoreMesh(core_axis_name="c", subcore_axis_name="s")

    @functools.partial(
        pl.kernel,
        out_type=jax.ShapeDtypeStruct((B, N, _CW), jnp.float32),
        mesh=mesh,
        scratch_types=[
            pltpu.VMEM_SHARED((trows, _CW), jnp.float32),
            pltpu.VMEM((toks_per_tile,), jnp.int32),
            pltpu.VMEM((toks_per_tile,), jnp.int32),
            pltpu.VMEM((2, CH), jnp.int32),
            pltpu.VMEM((2, CH, _CW), jnp.float32),
            pltpu.VMEM((CH, _LANES), jnp.float32),
            pltpu.SemaphoreType.DMA,
            pltpu.SemaphoreType.DMA,
            pltpu.SemaphoreType.DMA,
            pltpu.SemaphoreType.DMA,
        ],
    )
    def body(m_hbm, ig_hbm, is_hbm, w_hbm, out_hbm, tab_sh, ig_all, is_all,
             is_c, gbuf, w_all, sem0, sem1, sem2, sem3):
        cid = lax.axis_index("c")
        sid = lax.axis_index("s")
        base = cid * half
        row0 = sid * rows_tile
        tok0 = sid * toks_per_tile
        sems = (sem0, sem1)
        ssems = (sem2, sem3)
        zrows = 16

        def fetch(ch, slot):
            idx = ig_all.at[pl.ds(ch * CH, CH)]
            return pltpu.async_copy(m_hbm.at[idx], gbuf.at[slot],
                                    sems[slot])

        for b in range(B):
            _zero_vmem(gbuf.at[1], zrows, _CW)
            for z in range(rows_tile // zrows):
                pltpu.sync_copy(gbuf.at[1, pl.ds(0, zrows)],
                                tab_sh.at[pl.ds(row0 + z * zrows, zrows)])
            pltpu.sync_copy(ig_hbm.at[b, pl.ds(tok0, toks_per_tile)], ig_all)
            pltpu.sync_copy(is_hbm.at[b, pl.ds(tok0, toks_per_tile)], is_all)
            plsc.subcore_barrier()

            dma = fetch(0, 0)
            sdma = [None, None]
            for ch in range(nchunks):
                slot = ch % 2
                pltpu.sync_copy(w_hbm.at[b, pl.ds(tok0 + ch * CH, CH)],
                                w_all)
                dma.wait()
                if ch + 1 < nchunks:
                    if sdma[1 - slot] is not None:
                        sdma[1 - slot].wait()
                    dma = fetch(ch + 1, 1 - slot)
                for k2 in range(CH // _CHUNK):
                    _build_scatter_idx(
                        is_all, is_c.at[slot].at[pl.ds(k2 * _CHUNK, _CHUNK)],
                        ch * CH + k2 * _CHUNK, base, half)
                @plsc.parallel_loop(0, CH, unroll=4)
                def wmul(t):
                    wv = w_all[t, :]
                    for cb in range(_CW // _LANES):
                        sl = pl.ds(cb * _LANES, _LANES)
                        gbuf[slot, t, sl] = gbuf[slot, t, sl] * wv

                sdma[slot] = pltpu.async_copy(
                    gbuf.at[slot], tab_sh.at[is_c.at[slot]], ssems[slot],
                    add=True)
            for s in sdma:
                if s is not None:
                    s.wait()
            plsc.subcore_barrier()

            pltpu.sync_copy(tab_sh.at[pl.ds(row0, rows_tile)],
                            out_hbm.at[b, pl.ds(base + row0, rows_tile)])

    return body(map_pad, idx_gather, idx_scatter, w_bc)


_PAD = 72

_TAPS = (
    (3, 65, True, 0),
    (2, 64, False, 1),
    (3, 64, False, 2),
    (1, 1, True, 3),
    (0, 0, False, 4),
    (1, 0, False, 5),
    (3, 1, True, 6),
    (2, 0, False, 7),
    (3, 0, False, 8),
)


def _tc_conv(table, w9, bias, awt, B, C, Cout, Po, N):
    def body(tab_ref, w_ref, b_ref, aw_ref, out_ref, wb_ref, planes_ref):
        wb_ref[0] = jnp.broadcast_to(jnp.reshape(aw_ref[0], (N, 1)),
                                     (N, _LANES))
        for q in range(4):
            sl = pl.ds(q * Po, Po)
            acc = tab_ref[0, sl, 0:C]
            c = tab_ref[0, sl, C:C + 1]
            planes_ref[q, pl.ds(_PAD, Po), :] = acc / (c + 1e-6)
            planes_ref[q, pl.ds(0, _PAD), :] = jnp.zeros((_PAD, C),
                                                         jnp.float32)

        colmask = (lax.broadcasted_iota(jnp.int32, (Po, 1), 0) % 64) != 0
        out = jnp.broadcast_to(b_ref[0][None, :], (Po, Cout))
        for (q, sh, msk, k) in _TAPS:
            x = planes_ref[q, pl.ds(_PAD - sh, Po), :]
            if msk:
                x = jnp.where(colmask, x, 0.0)
            out = out + jnp.dot(x, w_ref[k],
                                preferred_element_type=jnp.float32)
        lane = lax.broadcasted_iota(jnp.int32, (Po, _CW - Cout), 1)
        marker = jnp.where(lane == 0, 1.0, 0.0)
        out_ref[0, :, 0:Cout] = out
        out_ref[0, :, Cout:_CW] = marker

    return pl.pallas_call(
        body,
        grid=(B,),
        in_specs=[
            pl.BlockSpec((1, 4 * Po, _CW), lambda b: (b, 0, 0)),
            pl.BlockSpec((9, C, Cout), lambda b: (0, 0, 0)),
            pl.BlockSpec((1, Cout), lambda b: (0, 0)),
            pl.BlockSpec((1, 1, N), lambda b: (b, 0, 0)),
        ],
        out_specs=[
            pl.BlockSpec((1, Po, _CW), lambda b: (b, 0, 0)),
            pl.BlockSpec((1, N, _LANES), lambda b: (b, 0, 0)),
        ],
        out_shape=[
            jax.ShapeDtypeStruct((B, Po, _CW), jnp.float32),
            jax.ShapeDtypeStruct((B, N, _LANES), jnp.float32),
        ],
        scratch_shapes=[pltpu.VMEM((4, _PAD + Po, C), jnp.float32)],
    )(table, w9, bias, awt)


def _tc_final(xt3, table2, skip_wT, params, prev, g, B, N, C, Cout):
    def body(x_ref, t_ref, sw_ref, p_ref, *rest):
        out_ref, conf_ref = rest[-2], rest[-1]
        acc_t = jnp.transpose(t_ref[0, :, 0:Cout])
        nrm_t = jnp.transpose(t_ref[0, :, Cout:Cout + 1])
        xt = acc_t / (nrm_t + 1e-6)
        xt = xt + jnp.dot(sw_ref[...], x_ref[0],
                          preferred_element_type=jnp.float32)
        xt = xt * p_ref[0][:, None] + p_ref[1][:, None]
        conf = jnp.sum(xt * p_ref[2][:, None], axis=0, keepdims=True) \
            + p_ref[3, 0]
        out_ref[0] = jnp.maximum(xt, 0.0)
        conf_ref[0] = conf

    RD = 2048
    in_specs = [
        pl.BlockSpec((1, C, RD), lambda b, r: (2 * b + g, 0, r)),
        pl.BlockSpec((1, RD, _CW), lambda b, r: (b, r, 0)),
        pl.BlockSpec((Cout, C), lambda b, r: (0, 0)),
        pl.BlockSpec((4, Cout), lambda b, r: (0, 0)),
    ]
    inputs = [xt3, table2, skip_wT, params]
    aliases = {}
    if prev is not None:
        in_specs += [pl.BlockSpec(memory_space=pl.ANY),
                     pl.BlockSpec(memory_space=pl.ANY)]
        inputs += [prev[0], prev[1]]
        aliases = {4: 0, 5: 1}
    return pl.pallas_call(
        body,
        grid=(B // 2, N // RD),
        in_specs=in_specs,
        out_specs=[
            pl.BlockSpec((1, Cout, RD), lambda b, r: (2 * b + g, 0, r)),
            pl.BlockSpec((1, 1, RD), lambda b, r: (2 * b + g, 0, r)),
        ],
        out_shape=[
            jax.ShapeDtypeStruct((B, Cout, N), jnp.float32),
            jax.ShapeDtypeStruct((B, 1, N), jnp.float32),
        ],
        input_output_aliases=aliases,
    )(*inputs)


def kernel(x, loc_orig, idx_agg, agg_weight, H, W, conv_w, conv_b, skip_w,
           bn_gamma, bn_beta, conf_w, conf_b):
    B, N, C = x.shape
    Cout = conv_w.shape[0]
    Hs = Ws = int(math.isqrt(N))
    Ho, Wo = Hs // 2, Ws // 2

    scale = jnp.array([W, H], dtype=jnp.float32)
    loc = jnp.rint(0.5 * (loc_orig + 1.0) * scale - 0.5).astype(jnp.int32)
    ix = jnp.clip(loc[..., 0], 0, W - 1)
    iy = jnp.clip(loc[..., 1], 0, H - 1)
    idxp = ((iy & 1) * (2 * Ho * Wo) + (ix & 1) * (Ho * Wo)
            + (iy >> 1) * Wo + (ix >> 1)).astype(jnp.int32)

    scale2 = jnp.array([Wo, Ho], dtype=jnp.float32)
    loc2 = jnp.rint(0.5 * (loc_orig + 1.0) * scale2 - 0.5).astype(jnp.int32)
    ix2 = jnp.clip(loc2[..., 0], 0, Wo - 1)
    iy2 = jnp.clip(loc2[..., 1], 0, Ho - 1)
    idx2 = (iy2 * Wo + ix2).astype(jnp.int32)

    ia = idx_agg.astype(jnp.int32)
    ia_off = ia + (jnp.arange(B, dtype=jnp.int32) * N)[:, None]
    idx2_off = idx2 + (jnp.arange(B, dtype=jnp.int32) * (Ho * Wo))[:, None]

    x_pad = jnp.concatenate([
        x,
        jnp.ones((B, N, 1), jnp.float32),
        jnp.zeros((B, N, _CW - C - 1), jnp.float32),
    ], axis=2).reshape(B * N, _CW)
    xt3 = x.transpose(0, 2, 1)
    awt = agg_weight.transpose(0, 2, 1)
    skip_wT = skip_w.T
    w9 = conv_w.transpose(2, 3, 1, 0).reshape(9, C, Cout)
    bias = conv_b.reshape(1, Cout)
    inv = 1.0 / jnp.sqrt(jnp.float32(1.0 + 1e-5))
    params = jnp.stack([
        bn_gamma * inv,
        bn_beta,
        conf_w[:, 0],
        jnp.broadcast_to(conf_b, (Cout,)),
    ])

    PG = 2
    prev = None
    for g in range(B // PG):
        sl = slice(g, None, 2)
        table1 = _sc_token2map(x_pad, ia_off[sl], idxp[sl], PG, N, Hs * Ws)
        map2, w_bc = _tc_conv(table1, w9, bias, awt[sl], PG, C, Cout,
                              Ho * Wo, N)
        m_pad = map2.reshape(PG * Ho * Wo, _CW)
        idx2_g = idx2[sl] + (jnp.arange(PG, dtype=jnp.int32)
                             * (Ho * Wo))[:, None]
        table2 = _sc_map2token(m_pad, idx2_g, ia[sl], w_bc, PG, N)
        prev = _tc_final(xt3, table2, skip_wT, params, prev, g,
                         B, N, C, Cout)

    return prev[0].transpose(0, 2, 1), prev[1].transpose(0, 2, 1)
